# baseline (speedup 1.0000x reference)
import jax
import jax.numpy as jnp
from jax import lax
from jax.experimental import pallas as pl
from jax.experimental.pallas import tpu as pltpu
from jax.experimental.pallas import tpu_sc as plsc

T = 2048
E = 64
D = 768
H = 768
CHUNK = 32
TPAD = T + E * 8 + CHUNK

NC = 2
NS = 16
NW = NC * NS
ROWS_W = T // NW


def _router_body(x_ref, rw_ref, rb_ref, dest_ref, starts_ref, counts_ref):
    logits = jnp.dot(x_ref[...], rw_ref[...],
                     preferred_element_type=jnp.float32) + rb_ref[...]
    m = jnp.max(logits, axis=1, keepdims=True)
    col = lax.broadcasted_iota(jnp.int32, (T, E), 1)
    eid = jnp.min(jnp.where(logits == m, col, E), axis=1, keepdims=True)
    oh = (col == eid).astype(jnp.int32)
    c = oh
    sh = 1
    while sh < T:
        c = c + jnp.concatenate(
            [jnp.zeros((sh, E), jnp.int32), c[:T - sh, :]], axis=0)
        sh *= 2
    counts = c[T - 1:T, :]
    padded = (counts + 7) & (-8)
    pc = padded
    sh = 1
    while sh < E:
        pc = pc + jnp.concatenate(
            [jnp.zeros((1, sh), jnp.int32), pc[:, :E - sh]], axis=1)
        sh *= 2
    starts = pc - padded
    rank = jnp.sum(oh * c, axis=1) - 1
    seg_start = jnp.sum(oh * starts, axis=1)
    dest_ref[...] = seg_start + rank
    starts_ref[...] = starts.reshape(E)
    counts_ref[...] = counts.reshape(E)


SCCH = 4
CW = ROWS_W // SCCH


def _dispatch_body(x_hbm, dest_hbm, out_hbm, idx_v, rows_v, sem_i, sem_a,
                   sem_b, sem_s):
    wid = lax.axis_index("s") * NC + lax.axis_index("c")
    base = wid * ROWS_W
    ih = [pltpu.async_copy(dest_hbm.at[pl.ds(base + k * CW, CW)],
                           idx_v.at[k], sem_i) for k in range(SCCH)]
    sems = [sem_a, sem_b]
    lh = [pltpu.async_copy(x_hbm.at[pl.ds(base + k * CW, CW)],
                           rows_v.at[k], sems[k % 2]) for k in range(SCCH)]
    for h in ih:
        h.wait()
    sh = []
    for k in range(SCCH):
        lh[k].wait()
        sh.append(pltpu.async_copy(rows_v.at[k], out_hbm.at[idx_v.at[k]],
                                   sem_s))
    for h in sh:
        h.wait()


def _undispatch_body(y_hbm, dest_hbm, out_hbm, idx_v, rows_v, sem_i, sem_a,
                     sem_b, sem_s):
    wid = lax.axis_index("s") * NC + lax.axis_index("c")
    base = wid * ROWS_W
    ih = [pltpu.async_copy(dest_hbm.at[pl.ds(base + k * CW, CW)],
                           idx_v.at[k], sem_i) for k in range(SCCH)]
    for h in ih:
        h.wait()
    sems = [sem_a, sem_b]
    gh = [pltpu.async_copy(y_hbm.at[idx_v.at[k]], rows_v.at[k], sems[k % 2])
          for k in range(SCCH)]
    sh = []
    for k in range(SCCH):
        gh[k].wait()
        sh.append(pltpu.async_copy(
            rows_v.at[k], out_hbm.at[pl.ds(base + k * CW, CW)], sem_s))
    for h in sh:
        h.wait()


EB = 4


def _ffn_body(starts_ref, counts_ref, xs_ref, w1_ref, b1_ref, w2_ref, b2_ref,
              out_ref, ys_v, pend_ref, sem_o):
    g = pl.program_id(0)

    @pl.when(g == 0)
    def _():
        pend_ref[0] = 0

    def wait_prev():
        @pl.when(pend_ref[0] == 1)
        def _():
            pltpu.make_async_copy(ys_v.at[pl.ds(0, CHUNK)],
                                  out_ref.at[pl.ds(0, CHUNK)], sem_o).wait()
        pend_ref[0] = 0

    for j in range(EB):
        e = g * EB + j
        start = starts_ref[e]
        n = counts_ref[e]
        nchunks = (n + CHUNK - 1) // CHUNK
        w1 = w1_ref[j]
        w2 = w2_ref[j]
        b1 = b1_ref[j]
        b2 = b2_ref[j]

        def body(i, carry):
            s = pl.multiple_of(start + i * CHUNK, 8)
            xb = xs_ref[pl.ds(s, CHUNK), :]
            h = jnp.maximum(
                jnp.dot(xb, w1, preferred_element_type=jnp.float32) + b1, 0.0)
            y = jnp.dot(h, w2, preferred_element_type=jnp.float32) + b2
            wait_prev()
            ys_v[pl.ds(s, CHUNK), :] = y
            pltpu.make_async_copy(ys_v.at[pl.ds(s, CHUNK)],
                                  out_ref.at[pl.ds(s, CHUNK)], sem_o).start()
            pend_ref[0] = 1
            return carry

        lax.fori_loop(0, nchunks, body, 0)

    @pl.when(g == (E // EB) - 1)
    def _():
        wait_prev()


def kernel(x, router_W, router_b, W1, b1, W2, b2):
    b_, s_, d_ = x.shape
    flat = x.reshape(T, D)

    dest1, starts, counts = pl.pallas_call(
        _router_body,
        out_shape=[
            jax.ShapeDtypeStruct((T,), jnp.int32),
            jax.ShapeDtypeStruct((E,), jnp.int32),
            jax.ShapeDtypeStruct((E,), jnp.int32),
        ],
    )(flat, router_W, router_b.reshape(1, E))

    mesh = plsc.VectorSubcoreMesh(
        core_axis_name="c", subcore_axis_name="s",
        num_cores=NC, num_subcores=NS)
    sc_scratch = [
        pltpu.VMEM((SCCH, CW), jnp.int32),
        pltpu.VMEM((SCCH, CW, D), jnp.float32),
        pltpu.SemaphoreType.DMA,
        pltpu.SemaphoreType.DMA,
        pltpu.SemaphoreType.DMA,
        pltpu.SemaphoreType.DMA,
    ]
    xs = pl.kernel(
        _dispatch_body,
        out_type=jax.ShapeDtypeStruct((TPAD, D), jnp.float32),
        mesh=mesh,
        scratch_types=sc_scratch,
    )(flat, dest1)

    grid_spec = pltpu.PrefetchScalarGridSpec(
        num_scalar_prefetch=2,
        grid=(E // EB,),
        in_specs=[
            pl.BlockSpec((TPAD, D), lambda e, s0, s1: (0, 0)),
            pl.BlockSpec((EB, D, H), lambda e, s0, s1: (e, 0, 0)),
            pl.BlockSpec((EB, 1, H), lambda e, s0, s1: (e, 0, 0)),
            pl.BlockSpec((EB, H, D), lambda e, s0, s1: (e, 0, 0)),
            pl.BlockSpec((EB, 1, D), lambda e, s0, s1: (e, 0, 0)),
        ],
        out_specs=pl.BlockSpec(memory_space=pltpu.HBM),
        scratch_shapes=[
            pltpu.VMEM((TPAD, D), jnp.float32),
            pltpu.SMEM((1,), jnp.int32),
            pltpu.SemaphoreType.DMA,
        ],
    )
    ys = pl.pallas_call(
        _ffn_body,
        grid_spec=grid_spec,
        out_shape=jax.ShapeDtypeStruct((TPAD, D), jnp.float32),
    )(starts, counts, xs, W1,
      b1.reshape(E, 1, H), W2, b2.reshape(E, 1, D))

    out = pl.kernel(
        _undispatch_body,
        out_type=jax.ShapeDtypeStruct((T, D), jnp.float32),
        mesh=mesh,
        scratch_types=sc_scratch,
    )(ys, dest1)
    return out.reshape(b_, s_, d_)


# speedup vs baseline: 1.0198x; 1.0198x over previous
import jax
import jax.numpy as jnp
from jax import lax
from jax.experimental import pallas as pl
from jax.experimental.pallas import tpu as pltpu
from jax.experimental.pallas import tpu_sc as plsc

T = 2048
E = 64
D = 768
H = 768
CHUNK = 64
TPAD = T + E * 8 + CHUNK

NC = 2
NS = 16
NW = NC * NS
ROWS_W = T // NW


RB = 4
RBS = T // RB


def _router_body(x_ref, rw_ref, rb_ref, dest_ref, starts_ref, counts_ref,
                 oh_v):
    g = pl.program_id(0)
    logits = jnp.dot(x_ref[...], rw_ref[...],
                     preferred_element_type=jnp.float32) + rb_ref[...]
    m = jnp.max(logits, axis=1, keepdims=True)
    col = lax.broadcasted_iota(jnp.int32, (RBS, E), 1)
    eid = jnp.min(jnp.where(logits == m, col, E), axis=1, keepdims=True)
    oh_v[pl.ds(g * RBS, RBS), :] = (col == eid).astype(jnp.int32)

    @pl.when(g == RB - 1)
    def _():
        oh = oh_v[...]
        c = oh
        sh = 1
        while sh < T:
            c = c + jnp.concatenate(
                [jnp.zeros((sh, E), jnp.int32), c[:T - sh, :]], axis=0)
            sh *= 2
        counts = c[T - 1:T, :]
        padded = (counts + 7) & (-8)
        pc = padded
        sh = 1
        while sh < E:
            pc = pc + jnp.concatenate(
                [jnp.zeros((1, sh), jnp.int32), pc[:, :E - sh]], axis=1)
            sh *= 2
        starts = pc - padded
        rank = jnp.sum(oh * c, axis=1) - 1
        seg_start = jnp.sum(oh * starts, axis=1)
        dest_ref[...] = seg_start + rank
        starts_ref[...] = starts.reshape(E)
        counts_ref[...] = counts.reshape(E)


SCCH = 4
CW = ROWS_W // SCCH


def _dispatch_body(x_hbm, dest_hbm, out_hbm, idx_v, rows_v, sem_i, sem_a,
                   sem_b, sem_s):
    wid = lax.axis_index("s") * NC + lax.axis_index("c")
    base = wid * ROWS_W
    ih = [pltpu.async_copy(dest_hbm.at[pl.ds(base + k * CW, CW)],
                           idx_v.at[k], sem_i) for k in range(SCCH)]
    sems = [sem_a, sem_b]
    lh = [pltpu.async_copy(x_hbm.at[pl.ds(base + k * CW, CW)],
                           rows_v.at[k], sems[k % 2]) for k in range(SCCH)]
    for h in ih:
        h.wait()
    sh = []
    for k in range(SCCH):
        lh[k].wait()
        sh.append(pltpu.async_copy(rows_v.at[k], out_hbm.at[idx_v.at[k]],
                                   sem_s))
    for h in sh:
        h.wait()


def _undispatch_body(y_hbm, dest_hbm, out_hbm, idx_v, rows_v, sem_i, sem_a,
                     sem_b, sem_s):
    wid = lax.axis_index("s") * NC + lax.axis_index("c")
    base = wid * ROWS_W
    ih = [pltpu.async_copy(dest_hbm.at[pl.ds(base + k * CW, CW)],
                           idx_v.at[k], sem_i) for k in range(SCCH)]
    for h in ih:
        h.wait()
    sems = [sem_a, sem_b]
    gh = [pltpu.async_copy(y_hbm.at[idx_v.at[k]], rows_v.at[k], sems[k % 2])
          for k in range(SCCH)]
    sh = []
    for k in range(SCCH):
        gh[k].wait()
        sh.append(pltpu.async_copy(
            rows_v.at[k], out_hbm.at[pl.ds(base + k * CW, CW)], sem_s))
    for h in sh:
        h.wait()


EB = 4


def _ffn_body(starts_ref, counts_ref, xs_ref, w1_ref, b1_ref, w2_ref, b2_ref,
              out_ref, ys_v, pend_ref, sem_o):
    g = pl.program_id(0)

    @pl.when(g == 0)
    def _():
        pend_ref[0] = 0

    def wait_prev():
        @pl.when(pend_ref[0] == 1)
        def _():
            pltpu.make_async_copy(ys_v.at[pl.ds(0, CHUNK)],
                                  out_ref.at[pl.ds(0, CHUNK)], sem_o).wait()
        pend_ref[0] = 0

    for j in range(EB):
        e = g * EB + j
        start = starts_ref[e]
        n = counts_ref[e]
        nchunks = (n + CHUNK - 1) // CHUNK
        w1 = w1_ref[j]
        w2 = w2_ref[j]
        b1 = b1_ref[j]
        b2 = b2_ref[j]

        def body(i, carry):
            s = pl.multiple_of(start + i * CHUNK, 8)
            xb = xs_ref[pl.ds(s, CHUNK), :]
            h = jnp.maximum(
                jnp.dot(xb, w1, preferred_element_type=jnp.float32) + b1, 0.0)
            y = jnp.dot(h, w2, preferred_element_type=jnp.float32) + b2
            wait_prev()
            ys_v[pl.ds(s, CHUNK), :] = y
            pltpu.make_async_copy(ys_v.at[pl.ds(s, CHUNK)],
                                  out_ref.at[pl.ds(s, CHUNK)], sem_o).start()
            pend_ref[0] = 1
            return carry

        lax.fori_loop(0, nchunks, body, 0)

    @pl.when(g == (E // EB) - 1)
    def _():
        wait_prev()


def kernel(x, router_W, router_b, W1, b1, W2, b2):
    b_, s_, d_ = x.shape
    flat = x.reshape(T, D)

    dest1, starts, counts = pl.pallas_call(
        _router_body,
        grid=(RB,),
        in_specs=[
            pl.BlockSpec((RBS, D), lambda g: (g, 0)),
            pl.BlockSpec((D, E), lambda g: (0, 0)),
            pl.BlockSpec((1, E), lambda g: (0, 0)),
        ],
        out_specs=[
            pl.BlockSpec((T,), lambda g: (0,)),
            pl.BlockSpec((E,), lambda g: (0,)),
            pl.BlockSpec((E,), lambda g: (0,)),
        ],
        out_shape=[
            jax.ShapeDtypeStruct((T,), jnp.int32),
            jax.ShapeDtypeStruct((E,), jnp.int32),
            jax.ShapeDtypeStruct((E,), jnp.int32),
        ],
        scratch_shapes=[pltpu.VMEM((T, E), jnp.int32)],
    )(flat, router_W, router_b.reshape(1, E))

    mesh = plsc.VectorSubcoreMesh(
        core_axis_name="c", subcore_axis_name="s",
        num_cores=NC, num_subcores=NS)
    sc_scratch = [
        pltpu.VMEM((SCCH, CW), jnp.int32),
        pltpu.VMEM((SCCH, CW, D), jnp.float32),
        pltpu.SemaphoreType.DMA,
        pltpu.SemaphoreType.DMA,
        pltpu.SemaphoreType.DMA,
        pltpu.SemaphoreType.DMA,
    ]
    xs = pl.kernel(
        _dispatch_body,
        out_type=jax.ShapeDtypeStruct((TPAD, D), jnp.float32),
        mesh=mesh,
        scratch_types=sc_scratch,
    )(flat, dest1)

    grid_spec = pltpu.PrefetchScalarGridSpec(
        num_scalar_prefetch=2,
        grid=(E // EB,),
        in_specs=[
            pl.BlockSpec((TPAD, D), lambda e, s0, s1: (0, 0)),
            pl.BlockSpec((EB, D, H), lambda e, s0, s1: (e, 0, 0)),
            pl.BlockSpec((EB, 1, H), lambda e, s0, s1: (e, 0, 0)),
            pl.BlockSpec((EB, H, D), lambda e, s0, s1: (e, 0, 0)),
            pl.BlockSpec((EB, 1, D), lambda e, s0, s1: (e, 0, 0)),
        ],
        out_specs=pl.BlockSpec(memory_space=pltpu.HBM),
        scratch_shapes=[
            pltpu.VMEM((TPAD, D), jnp.float32),
            pltpu.SMEM((1,), jnp.int32),
            pltpu.SemaphoreType.DMA,
        ],
    )
    ys = pl.pallas_call(
        _ffn_body,
        grid_spec=grid_spec,
        out_shape=jax.ShapeDtypeStruct((TPAD, D), jnp.float32),
    )(starts, counts, xs, W1,
      b1.reshape(E, 1, H), W2, b2.reshape(E, 1, D))

    out = pl.kernel(
        _undispatch_body,
        out_type=jax.ShapeDtypeStruct((T, D), jnp.float32),
        mesh=mesh,
        scratch_types=sc_scratch,
    )(ys, dest1)
    return out.reshape(b_, s_, d_)


# speedup vs baseline: 1.0273x; 1.0074x over previous
import jax
import jax.numpy as jnp
from jax import lax
from jax.experimental import pallas as pl
from jax.experimental.pallas import tpu as pltpu
from jax.experimental.pallas import tpu_sc as plsc

T = 2048
E = 64
D = 768
H = 768
CHUNK = 64
TPAD = T + E * 8 + CHUNK

NC = 2
NS = 16
NW = NC * NS
ROWS_W = T // NW


def _router_body(x_ref, rw_ref, rb_ref, dest_ref, starts_ref, counts_ref):
    logits = jnp.dot(x_ref[...], rw_ref[...],
                     preferred_element_type=jnp.float32) + rb_ref[...]
    m = jnp.max(logits, axis=1, keepdims=True)
    col = lax.broadcasted_iota(jnp.int32, (T, E), 1)
    eid = jnp.min(jnp.where(logits == m, col, E), axis=1, keepdims=True)
    oh = (col == eid).astype(jnp.int32)
    c = oh
    sh = 1
    while sh < T:
        c = c + jnp.concatenate(
            [jnp.zeros((sh, E), jnp.int32), c[:T - sh, :]], axis=0)
        sh *= 2
    counts = c[T - 1:T, :]
    padded = (counts + 7) & (-8)
    pc = padded
    sh = 1
    while sh < E:
        pc = pc + jnp.concatenate(
            [jnp.zeros((1, sh), jnp.int32), pc[:, :E - sh]], axis=1)
        sh *= 2
    starts = pc - padded
    rank = jnp.sum(oh * c, axis=1) - 1
    seg_start = jnp.sum(oh * starts, axis=1)
    dest_ref[...] = seg_start + rank
    starts_ref[...] = starts.reshape(E)
    counts_ref[...] = counts.reshape(E)


SCCH = 4
CW = ROWS_W // SCCH


def _dispatch_body(x_hbm, dest_hbm, out_hbm, idx_v, rows_v, sem_i, sem_a,
                   sem_b, sem_s):
    wid = lax.axis_index("s") * NC + lax.axis_index("c")
    base = wid * ROWS_W
    ih = [pltpu.async_copy(dest_hbm.at[pl.ds(base + k * CW, CW)],
                           idx_v.at[k], sem_i) for k in range(SCCH)]
    sems = [sem_a, sem_b]
    lh = [pltpu.async_copy(x_hbm.at[pl.ds(base + k * CW, CW)],
                           rows_v.at[k], sems[k % 2]) for k in range(SCCH)]
    for h in ih:
        h.wait()
    sh = []
    for k in range(SCCH):
        lh[k].wait()
        sh.append(pltpu.async_copy(rows_v.at[k], out_hbm.at[idx_v.at[k]],
                                   sem_s))
    for h in sh:
        h.wait()


def _undispatch_body(y_hbm, dest_hbm, out_hbm, idx_v, rows_v, sem_i, sem_a,
                     sem_b, sem_s):
    wid = lax.axis_index("s") * NC + lax.axis_index("c")
    base = wid * ROWS_W
    ih = [pltpu.async_copy(dest_hbm.at[pl.ds(base + k * CW, CW)],
                           idx_v.at[k], sem_i) for k in range(SCCH)]
    for h in ih:
        h.wait()
    sems = [sem_a, sem_b]
    gh = [pltpu.async_copy(y_hbm.at[idx_v.at[k]], rows_v.at[k], sems[k % 2])
          for k in range(SCCH)]
    sh = []
    for k in range(SCCH):
        gh[k].wait()
        sh.append(pltpu.async_copy(
            rows_v.at[k], out_hbm.at[pl.ds(base + k * CW, CW)], sem_s))
    for h in sh:
        h.wait()


EB = 4


def _ffn_body(starts_ref, counts_ref, xs_ref, w1_ref, b1_ref, w2_ref, b2_ref,
              out_ref, ys_v, pend_ref, sem_o):
    g = pl.program_id(0)

    @pl.when(g == 0)
    def _():
        pend_ref[0] = 0

    def wait_prev():
        @pl.when(pend_ref[0] == 1)
        def _():
            pltpu.make_async_copy(ys_v.at[pl.ds(0, CHUNK)],
                                  out_ref.at[pl.ds(0, CHUNK)], sem_o).wait()
        pend_ref[0] = 0

    for j in range(EB):
        e = g * EB + j
        start = starts_ref[e]
        n = counts_ref[e]
        nchunks = (n + CHUNK - 1) // CHUNK
        w1 = w1_ref[j]
        w2 = w2_ref[j]
        b1 = b1_ref[j]
        b2 = b2_ref[j]

        def body(i, carry):
            s = pl.multiple_of(start + i * CHUNK, 8)
            xb = xs_ref[pl.ds(s, CHUNK), :]
            h = jnp.maximum(
                jnp.dot(xb, w1, preferred_element_type=jnp.float32) + b1, 0.0)
            y = jnp.dot(h, w2, preferred_element_type=jnp.float32) + b2
            wait_prev()
            ys_v[pl.ds(s, CHUNK), :] = y
            pltpu.make_async_copy(ys_v.at[pl.ds(s, CHUNK)],
                                  out_ref.at[pl.ds(s, CHUNK)], sem_o).start()
            pend_ref[0] = 1
            return carry

        lax.fori_loop(0, nchunks, body, 0)

    @pl.when(g == (E // EB) - 1)
    def _():
        wait_prev()


def kernel(x, router_W, router_b, W1, b1, W2, b2):
    b_, s_, d_ = x.shape
    flat = x.reshape(T, D)

    dest1, starts, counts = pl.pallas_call(
        _router_body,
        out_shape=[
            jax.ShapeDtypeStruct((T,), jnp.int32),
            jax.ShapeDtypeStruct((E,), jnp.int32),
            jax.ShapeDtypeStruct((E,), jnp.int32),
        ],
    )(flat, router_W, router_b.reshape(1, E))

    mesh = plsc.VectorSubcoreMesh(
        core_axis_name="c", subcore_axis_name="s",
        num_cores=NC, num_subcores=NS)
    sc_scratch = [
        pltpu.VMEM((SCCH, CW), jnp.int32),
        pltpu.VMEM((SCCH, CW, D), jnp.float32),
        pltpu.SemaphoreType.DMA,
        pltpu.SemaphoreType.DMA,
        pltpu.SemaphoreType.DMA,
        pltpu.SemaphoreType.DMA,
    ]
    xs = pl.kernel(
        _dispatch_body,
        out_type=jax.ShapeDtypeStruct((TPAD, D), jnp.float32),
        mesh=mesh,
        scratch_types=sc_scratch,
    )(flat, dest1)

    grid_spec = pltpu.PrefetchScalarGridSpec(
        num_scalar_prefetch=2,
        grid=(E // EB,),
        in_specs=[
            pl.BlockSpec((TPAD, D), lambda e, s0, s1: (0, 0)),
            pl.BlockSpec((EB, D, H), lambda e, s0, s1: (e, 0, 0)),
            pl.BlockSpec((EB, 1, H), lambda e, s0, s1: (e, 0, 0)),
            pl.BlockSpec((EB, H, D), lambda e, s0, s1: (e, 0, 0)),
            pl.BlockSpec((EB, 1, D), lambda e, s0, s1: (e, 0, 0)),
        ],
        out_specs=pl.BlockSpec(memory_space=pltpu.HBM),
        scratch_shapes=[
            pltpu.VMEM((TPAD, D), jnp.float32),
            pltpu.SMEM((1,), jnp.int32),
            pltpu.SemaphoreType.DMA,
        ],
    )
    ys = pl.pallas_call(
        _ffn_body,
        grid_spec=grid_spec,
        out_shape=jax.ShapeDtypeStruct((TPAD, D), jnp.float32),
    )(starts, counts, xs, W1,
      b1.reshape(E, 1, H), W2, b2.reshape(E, 1, D))

    out = pl.kernel(
        _undispatch_body,
        out_type=jax.ShapeDtypeStruct((T, D), jnp.float32),
        mesh=mesh,
        scratch_types=sc_scratch,
    )(ys, dest1)
    return out.reshape(b_, s_, d_)
